# baseline (speedup 1.0000x reference)
import jax
import jax.numpy as jnp
import numpy as np
from jax import lax
from jax.experimental import pallas as pl

_DIFF_RATIO = 0.01
_HUBER_DELTA = 0.03
_LAMBDA_ACCEL = 0.2
_KC = 6

_SENT = np.int32(0x7FFFFFFF)
_SIGN = np.int32(-2147483648)


def _map_keys(x):
    b = lax.bitcast_convert_type(x, jnp.int32)
    return jnp.where(b < 0, (~b) ^ _SIGN, b)


def _unmap_key(s):
    b = jnp.where(s >= 0, s, ~(s ^ _SIGN))
    return lax.bitcast_convert_type(b, jnp.float32)


def _fcount(pred):
    return jnp.sum(pred.astype(jnp.float32))


def _count16(arr, t_i32, strict=False):
    t16 = t_i32.astype(jnp.int16)
    cond = (arr < t16) if strict else (arr <= t16)
    a3 = cond.astype(jnp.bfloat16).reshape(32, -1, arr.shape[-1])
    part = jnp.sum(a3, axis=0)
    return jnp.sum(part.astype(jnp.float32))


def _select2(keys_a, keys_b, target_a, target_b):
    ka2 = keys_a.reshape(-1, keys_a.shape[-1])
    kb2 = keys_b.reshape(-1, keys_b.shape[-1])
    ha = (ka2 >> 16).astype(jnp.int16)
    hb = (kb2 >> 16).astype(jnp.int16)

    def step_hi(i, carry):
        pa, pb = carry
        bit = 31 - i
        low = (jnp.int32(1) << bit) - 1
        ca = _count16(ha, ((pa | low) ^ _SIGN) >> 16)
        cb = _count16(hb, ((pb | low) ^ _SIGN) >> 16)
        hi = low + 1
        pa = jnp.where(ca >= target_a, pa, pa | hi)
        pb = jnp.where(cb >= target_b, pb, pb | hi)
        return pa, pb

    pa, pb = lax.fori_loop(0, 16, step_hi, (jnp.int32(0), jnp.int32(0)))

    hsa = (pa ^ _SIGN) >> 16
    hsb = (pb ^ _SIGN) >> 16
    base_a = _count16(ha, hsa, strict=True)
    base_b = _count16(hb, hsb, strict=True)
    la = jnp.where(ha == hsa.astype(jnp.int16),
                   ((ka2 & 0xFFFF) - 32768).astype(jnp.int16), np.int16(32767))
    lb = jnp.where(hb == hsb.astype(jnp.int16),
                   ((kb2 & 0xFFFF) - 32768).astype(jnp.int16), np.int16(32767))

    def step_lo(i, carry):
        qa, qb = carry
        bit = 15 - i
        low = (jnp.int32(1) << bit) - 1
        ca = base_a + _count16(la, (qa | low) - 32768)
        cb = base_b + _count16(lb, (qb | low) - 32768)
        hi = low + 1
        qa = jnp.where(ca >= target_a, qa, qa | hi)
        qb = jnp.where(cb >= target_b, qb, qb | hi)
        return qa, qb

    qa, qb = lax.fori_loop(0, 16, step_lo, (jnp.int32(0), jnp.int32(0)))
    return ((pa | qa) ^ _SIGN), ((pb | qb) ^ _SIGN)


def _huber(x):
    quad = 0.5 * jnp.minimum(x, _HUBER_DELTA) ** 2 / _HUBER_DELTA
    lin = x - 0.5 * _HUBER_DELTA
    return jnp.where(x <= _HUBER_DELTA, quad, lin)


def _trimmed_loss(r, keys, tau_key, k):
    lt = keys < tau_key
    cnt_lt = _fcount(lt)
    s = jnp.sum(jnp.where(lt, _huber(r), 0.0))
    tau = lax.bitcast_convert_type(tau_key, jnp.float32)
    total = s + (k - cnt_lt) * _huber(tau)
    return jnp.where(k > 0, total / jnp.maximum(k, 1.0), 0.0)


def _body(p_ref, g_ref, m_ref, pv_ref, scal_ref, curr_ref):
    P = p_ref[0]
    G = g_ref[0]
    Mb = m_ref[0] > 0.5
    Mf = Mb.astype(jnp.float32)

    MP = Mf * P
    a00 = jnp.sum(MP * P)
    a01 = jnp.sum(MP)
    a11 = jnp.sum(Mf)
    b0 = jnp.sum(MP * G)
    b1 = jnp.sum(Mf * G)
    det = a00 * a11 - a01 * a01
    denom = det + 1e-06
    nz = det != 0
    s = jnp.where(nz, (a11 * b0 - a01 * b1) / denom, 0.0)
    t = jnp.where(nz, (-a01 * b0 + a00 * b1) / denom, 0.0)

    gkey = jnp.where(Mb, _map_keys(G), _SENT)
    nf = a11

    def qrank(q):
        virt = jnp.maximum(q * (nf - 1.0), 0.0)
        prev = virt.astype(jnp.int32).astype(jnp.float32)
        gamma = virt - prev
        ni = jnp.minimum(prev + 1.0, jnp.maximum(nf - 1.0, 0.0))
        return prev, ni, gamma

    pi_lo, ni_lo, gam_lo = qrank(0.05)
    pi_hi, ni_hi, gam_hi = qrank(0.95)

    klo, khi = _select2(gkey, gkey, pi_lo + 1.0, pi_hi + 1.0)

    cle_lo = _fcount(gkey <= klo)
    cle_hi = _fcount(gkey <= khi)
    mingt_lo = jnp.min(jnp.where(gkey > klo, gkey, _SENT))
    mingt_hi = jnp.min(jnp.where(gkey > khi, gkey, _SENT))
    klo2 = jnp.where(cle_lo >= ni_lo + 1.0, klo, mingt_lo)
    khi2 = jnp.where(cle_hi >= ni_hi + 1.0, khi, mingt_hi)

    def qval(a_key, b_key, gamma):
        a = _unmap_key(a_key)
        b = _unmap_key(b_key)
        d = b - a
        return jnp.where(gamma >= 0.5, b - d * (1.0 - gamma), a + d * gamma)

    q_lo = qval(klo, klo2, gam_lo)
    q_hi = qval(khi, khi2, gam_hi)
    rng = q_hi - q_lo
    inf = np.float32(np.inf)
    ok = (nf > 0) & (jnp.abs(q_lo) < inf) & (jnp.abs(q_hi) < inf) & (rng > 0)
    rr = jnp.where(ok, jnp.maximum(rng, 1e-06), 1.0)
    th = _DIFF_RATIO * rr
    scale = jnp.maximum(rr, 1e-06)

    pa1 = s * P[1] + t
    pa0 = s * P[0] + t
    dpred = pa1 - pa0
    curr_ref[0, 0] = dpred
    dgt = G[1] - G[0]
    static = Mb[1] & Mb[0] & (jnp.abs(dgt) < th)
    n = _fcount(static)
    k = ((_KC * n) / 10.0).astype(jnp.int32).astype(jnp.float32)

    r_d = jnp.abs(dpred - dgt) / scale
    r_a = jnp.abs(dpred - pv_ref[0, 0]) / scale
    kd = jnp.where(static, lax.bitcast_convert_type(r_d, jnp.int32), _SENT)
    ka = jnp.where(static, lax.bitcast_convert_type(r_a, jnp.int32), _SENT)
    tgt = jnp.maximum(k, 1.0)
    tau_d, tau_a = _select2(kd, ka, tgt, tgt)

    loss_d = _trimmed_loss(r_d, kd, tau_d, k)
    loss_a = _trimmed_loss(r_a, ka, tau_a, k)

    lane = lax.broadcasted_iota(jnp.int32, (1, 1, 128), 2)
    row = jnp.where(lane == 0, loss_d,
                    jnp.where(lane == 1, loss_a,
                              jnp.where(lane == 2, rr, 0.0)))
    scal_ref[...] = row


@jax.jit
def kernel(pred_pair, gt_pair, mask_pair, prev_pred_grad):
    B, T, H, W = pred_pair.shape
    grid = (B,)
    scal, curr = pl.pallas_call(
        _body,
        grid=grid,
        in_specs=[
            pl.BlockSpec((1, 2, H, W), lambda b: (b, 0, 0, 0)),
            pl.BlockSpec((1, 2, H, W), lambda b: (b, 0, 0, 0)),
            pl.BlockSpec((1, 2, H, W), lambda b: (b, 0, 0, 0)),
            pl.BlockSpec((1, 1, H, W), lambda b: (b, 0, 0, 0)),
        ],
        out_specs=[
            pl.BlockSpec((1, 1, 128), lambda b: (b, 0, 0)),
            pl.BlockSpec((1, 1, H, W), lambda b: (b, 0, 0, 0)),
        ],
        out_shape=[
            jax.ShapeDtypeStruct((B, 1, 128), jnp.float32),
            jax.ShapeDtypeStruct((B, 1, H, W), jnp.float32),
        ],
    )(pred_pair, gt_pair, mask_pair, prev_pred_grad)
    loss_data = jnp.mean(scal[:, 0, 0])
    loss_acc = jnp.mean(scal[:, 0, 1])
    mean_rr = jnp.mean(scal[:, 0, 2])
    total = loss_data + _LAMBDA_ACCEL * loss_acc
    return (total, loss_data, loss_acc, mean_rr, curr)


# speedup vs baseline: 2.2360x; 2.2360x over previous
import jax
import jax.numpy as jnp
import numpy as np
from jax import lax
from jax.experimental import pallas as pl

_DIFF_RATIO = 0.01
_HUBER_DELTA = 0.03
_LAMBDA_ACCEL = 0.2
_KC = 6

_SENT = np.int32(0x7FFFFFFF)
_SIGN = np.int32(-2147483648)


def _map_keys(x):
    b = lax.bitcast_convert_type(x, jnp.int32)
    return jnp.where(b < 0, (~b) ^ _SIGN, b)


def _unmap_key(s):
    b = jnp.where(s >= 0, s, ~(s ^ _SIGN))
    return lax.bitcast_convert_type(b, jnp.float32)


def _fcount(pred):
    return jnp.sum(pred.astype(jnp.float32))


def _count16(arr, t_i32, strict=False):
    t16 = t_i32.astype(jnp.int16)
    cond = (arr < t16) if strict else (arr <= t16)
    axes = tuple(range(arr.ndim - 1))
    part = jnp.sum(jnp.where(cond, np.int16(1), np.int16(0)), axis=axes)
    return jnp.sum(part.astype(jnp.float32))


def _select2(keys_a, keys_b, target_a, target_b):
    ka2 = keys_a.reshape(-1, keys_a.shape[-1])
    kb2 = keys_b.reshape(-1, keys_b.shape[-1])
    ha = (ka2 >> 16).astype(jnp.int16)
    hb = (kb2 >> 16).astype(jnp.int16)

    def step_hi(i, carry):
        pa, pb = carry
        bit = 31 - i
        low = (jnp.int32(1) << bit) - 1
        ca = _count16(ha, ((pa | low) ^ _SIGN) >> 16)
        cb = _count16(hb, ((pb | low) ^ _SIGN) >> 16)
        hi = low + 1
        pa = jnp.where(ca >= target_a, pa, pa | hi)
        pb = jnp.where(cb >= target_b, pb, pb | hi)
        return pa, pb

    pa, pb = lax.fori_loop(0, 16, step_hi, (jnp.int32(0), jnp.int32(0)))

    hsa = (pa ^ _SIGN) >> 16
    hsb = (pb ^ _SIGN) >> 16
    base_a = _count16(ha, hsa, strict=True)
    base_b = _count16(hb, hsb, strict=True)
    la = jnp.where(ha == hsa.astype(jnp.int16),
                   ((ka2 & 0xFFFF) - 32768).astype(jnp.int16), np.int16(32767))
    lb = jnp.where(hb == hsb.astype(jnp.int16),
                   ((kb2 & 0xFFFF) - 32768).astype(jnp.int16), np.int16(32767))

    def step_lo(i, carry):
        qa, qb = carry
        bit = 15 - i
        low = (jnp.int32(1) << bit) - 1
        ca = base_a + _count16(la, (qa | low) - 32768)
        cb = base_b + _count16(lb, (qb | low) - 32768)
        hi = low + 1
        qa = jnp.where(ca >= target_a, qa, qa | hi)
        qb = jnp.where(cb >= target_b, qb, qb | hi)
        return qa, qb

    qa, qb = lax.fori_loop(0, 16, step_lo, (jnp.int32(0), jnp.int32(0)))
    return ((pa | qa) ^ _SIGN), ((pb | qb) ^ _SIGN)


def _huber(x):
    quad = 0.5 * jnp.minimum(x, _HUBER_DELTA) ** 2 / _HUBER_DELTA
    lin = x - 0.5 * _HUBER_DELTA
    return jnp.where(x <= _HUBER_DELTA, quad, lin)


def _trimmed_loss(r, keys, tau_key, k):
    lt = keys < tau_key
    cnt_lt = _fcount(lt)
    s = jnp.sum(jnp.where(lt, _huber(r), 0.0))
    tau = lax.bitcast_convert_type(tau_key, jnp.float32)
    total = s + (k - cnt_lt) * _huber(tau)
    return jnp.where(k > 0, total / jnp.maximum(k, 1.0), 0.0)


def _body(p_ref, g_ref, m_ref, pv_ref, scal_ref, curr_ref):
    P = p_ref[0]
    G = g_ref[0]
    Mb = m_ref[0] > 0.5
    Mf = Mb.astype(jnp.float32)

    MP = Mf * P
    a00 = jnp.sum(MP * P)
    a01 = jnp.sum(MP)
    a11 = jnp.sum(Mf)
    b0 = jnp.sum(MP * G)
    b1 = jnp.sum(Mf * G)
    det = a00 * a11 - a01 * a01
    denom = det + 1e-06
    nz = det != 0
    s = jnp.where(nz, (a11 * b0 - a01 * b1) / denom, 0.0)
    t = jnp.where(nz, (-a01 * b0 + a00 * b1) / denom, 0.0)

    gkey = jnp.where(Mb, _map_keys(G), _SENT)
    nf = a11

    def qrank(q):
        virt = jnp.maximum(q * (nf - 1.0), 0.0)
        prev = virt.astype(jnp.int32).astype(jnp.float32)
        gamma = virt - prev
        ni = jnp.minimum(prev + 1.0, jnp.maximum(nf - 1.0, 0.0))
        return prev, ni, gamma

    pi_lo, ni_lo, gam_lo = qrank(0.05)
    pi_hi, ni_hi, gam_hi = qrank(0.95)

    klo, khi = _select2(gkey, gkey, pi_lo + 1.0, pi_hi + 1.0)

    cle_lo = _fcount(gkey <= klo)
    cle_hi = _fcount(gkey <= khi)
    mingt_lo = jnp.min(jnp.where(gkey > klo, gkey, _SENT))
    mingt_hi = jnp.min(jnp.where(gkey > khi, gkey, _SENT))
    klo2 = jnp.where(cle_lo >= ni_lo + 1.0, klo, mingt_lo)
    khi2 = jnp.where(cle_hi >= ni_hi + 1.0, khi, mingt_hi)

    def qval(a_key, b_key, gamma):
        a = _unmap_key(a_key)
        b = _unmap_key(b_key)
        d = b - a
        return jnp.where(gamma >= 0.5, b - d * (1.0 - gamma), a + d * gamma)

    q_lo = qval(klo, klo2, gam_lo)
    q_hi = qval(khi, khi2, gam_hi)
    rng = q_hi - q_lo
    inf = np.float32(np.inf)
    ok = (nf > 0) & (jnp.abs(q_lo) < inf) & (jnp.abs(q_hi) < inf) & (rng > 0)
    rr = jnp.where(ok, jnp.maximum(rng, 1e-06), 1.0)
    th = _DIFF_RATIO * rr
    scale = jnp.maximum(rr, 1e-06)

    pa1 = s * P[1] + t
    pa0 = s * P[0] + t
    dpred = pa1 - pa0
    curr_ref[0, 0] = dpred
    dgt = G[1] - G[0]
    static = Mb[1] & Mb[0] & (jnp.abs(dgt) < th)
    n = _fcount(static)
    k = ((_KC * n) / 10.0).astype(jnp.int32).astype(jnp.float32)

    r_d = jnp.abs(dpred - dgt) / scale
    r_a = jnp.abs(dpred - pv_ref[0, 0]) / scale
    kd = jnp.where(static, lax.bitcast_convert_type(r_d, jnp.int32), _SENT)
    ka = jnp.where(static, lax.bitcast_convert_type(r_a, jnp.int32), _SENT)
    tgt = jnp.maximum(k, 1.0)
    tau_d, tau_a = _select2(kd, ka, tgt, tgt)

    loss_d = _trimmed_loss(r_d, kd, tau_d, k)
    loss_a = _trimmed_loss(r_a, ka, tau_a, k)

    lane = lax.broadcasted_iota(jnp.int32, (1, 1, 128), 2)
    row = jnp.where(lane == 0, loss_d,
                    jnp.where(lane == 1, loss_a,
                              jnp.where(lane == 2, rr, 0.0)))
    scal_ref[...] = row


@jax.jit
def kernel(pred_pair, gt_pair, mask_pair, prev_pred_grad):
    B, T, H, W = pred_pair.shape
    grid = (B,)
    scal, curr = pl.pallas_call(
        _body,
        grid=grid,
        in_specs=[
            pl.BlockSpec((1, 2, H, W), lambda b: (b, 0, 0, 0)),
            pl.BlockSpec((1, 2, H, W), lambda b: (b, 0, 0, 0)),
            pl.BlockSpec((1, 2, H, W), lambda b: (b, 0, 0, 0)),
            pl.BlockSpec((1, 1, H, W), lambda b: (b, 0, 0, 0)),
        ],
        out_specs=[
            pl.BlockSpec((1, 1, 128), lambda b: (b, 0, 0)),
            pl.BlockSpec((1, 1, H, W), lambda b: (b, 0, 0, 0)),
        ],
        out_shape=[
            jax.ShapeDtypeStruct((B, 1, 128), jnp.float32),
            jax.ShapeDtypeStruct((B, 1, H, W), jnp.float32),
        ],
    )(pred_pair, gt_pair, mask_pair, prev_pred_grad)
    loss_data = jnp.mean(scal[:, 0, 0])
    loss_acc = jnp.mean(scal[:, 0, 1])
    mean_rr = jnp.mean(scal[:, 0, 2])
    total = loss_data + _LAMBDA_ACCEL * loss_acc
    return (total, loss_data, loss_acc, mean_rr, curr)


# speedup vs baseline: 2.5530x; 1.1418x over previous
import jax
import jax.numpy as jnp
import numpy as np
from jax import lax
from jax.experimental import pallas as pl

_DIFF_RATIO = 0.01
_HUBER_DELTA = 0.03
_LAMBDA_ACCEL = 0.2
_KC = 6

_SENT = np.int32(0x7FFFFFFF)
_SIGN = np.int32(-2147483648)

_BB = 2


def _map_keys(x):
    b = lax.bitcast_convert_type(x, jnp.int32)
    return jnp.where(b < 0, (~b) ^ _SIGN, b)


def _unmap_key(s):
    b = jnp.where(s >= 0, s, ~(s ^ _SIGN))
    return lax.bitcast_convert_type(b, jnp.float32)


def _fcount(pred):
    return jnp.sum(pred.astype(jnp.float32))


def _count16(arr, t_i32, strict=False):
    t16 = t_i32.astype(jnp.int16)
    cond = (arr < t16) if strict else (arr <= t16)
    axes = tuple(range(arr.ndim - 1))
    part = jnp.sum(jnp.where(cond, np.int16(1), np.int16(0)), axis=axes)
    return jnp.sum(part.astype(jnp.float32))


def _select_many(keys_list, targets):
    n = len(keys_list)
    k2 = [k.reshape(-1, k.shape[-1]) for k in keys_list]
    hs = [(k >> 16).astype(jnp.int16) for k in k2]

    def step_hi(i, ps):
        bit = 31 - i
        low = (jnp.int32(1) << bit) - 1
        cs = [_count16(hs[j], ((ps[j] | low) ^ _SIGN) >> 16) for j in range(n)]
        hi = low + 1
        return tuple(jnp.where(cs[j] >= targets[j], ps[j], ps[j] | hi)
                     for j in range(n))

    ps = lax.fori_loop(0, 16, step_hi, (jnp.int32(0),) * n)

    hstar = [(p ^ _SIGN) >> 16 for p in ps]
    bases = [_count16(hs[j], hstar[j], strict=True) for j in range(n)]
    los = [jnp.where(hs[j] == hstar[j].astype(jnp.int16),
                     ((k2[j] & 0xFFFF) - 32768).astype(jnp.int16),
                     np.int16(32767))
           for j in range(n)]

    def step_lo(i, qs):
        bit = 15 - i
        low = (jnp.int32(1) << bit) - 1
        cs = [bases[j] + _count16(los[j], (qs[j] | low) - 32768)
              for j in range(n)]
        hi = low + 1
        return tuple(jnp.where(cs[j] >= targets[j], qs[j], qs[j] | hi)
                     for j in range(n))

    qs = lax.fori_loop(0, 16, step_lo, (jnp.int32(0),) * n)
    return [(ps[j] | qs[j]) ^ _SIGN for j in range(n)]


def _huber(x):
    quad = 0.5 * jnp.minimum(x, _HUBER_DELTA) ** 2 / _HUBER_DELTA
    lin = x - 0.5 * _HUBER_DELTA
    return jnp.where(x <= _HUBER_DELTA, quad, lin)


def _trimmed_loss(r, keys, tau_key, k):
    lt = keys < tau_key
    cnt_lt = _fcount(lt)
    s = jnp.sum(jnp.where(lt, _huber(r), 0.0))
    tau = lax.bitcast_convert_type(tau_key, jnp.float32)
    total = s + (k - cnt_lt) * _huber(tau)
    return jnp.where(k > 0, total / jnp.maximum(k, 1.0), 0.0)


def _qrank(q, nf):
    virt = jnp.maximum(q * (nf - 1.0), 0.0)
    prev = virt.astype(jnp.int32).astype(jnp.float32)
    gamma = virt - prev
    ni = jnp.minimum(prev + 1.0, jnp.maximum(nf - 1.0, 0.0))
    return prev, ni, gamma


def _body(p_ref, g_ref, m_ref, pv_ref, scal_ref, curr_ref):
    P = [p_ref[j] for j in range(_BB)]
    G = [g_ref[j] for j in range(_BB)]
    Mb = [m_ref[j] > 0.5 for j in range(_BB)]

    st = []
    nf = []
    for j in range(_BB):
        Mf = Mb[j].astype(jnp.float32)
        MP = Mf * P[j]
        a00 = jnp.sum(MP * P[j])
        a01 = jnp.sum(MP)
        a11 = jnp.sum(Mf)
        b0 = jnp.sum(MP * G[j])
        b1 = jnp.sum(Mf * G[j])
        det = a00 * a11 - a01 * a01
        denom = det + 1e-06
        nz = det != 0
        s = jnp.where(nz, (a11 * b0 - a01 * b1) / denom, 0.0)
        t = jnp.where(nz, (-a01 * b0 + a00 * b1) / denom, 0.0)
        st.append((s, t))
        nf.append(a11)

    gkeys = [jnp.where(Mb[j], _map_keys(G[j]), _SENT) for j in range(_BB)]
    ranks = [( _qrank(0.05, nf[j]), _qrank(0.95, nf[j])) for j in range(_BB)]

    sel_keys, sel_tgts = [], []
    for j in range(_BB):
        (pi_lo, _, _), (pi_hi, _, _) = ranks[j]
        sel_keys += [gkeys[j], gkeys[j]]
        sel_tgts += [pi_lo + 1.0, pi_hi + 1.0]
    qsel = _select_many(sel_keys, sel_tgts)

    rrs, ths, scales = [], [], []
    for j in range(_BB):
        klo, khi = qsel[2 * j], qsel[2 * j + 1]
        (pi_lo, ni_lo, gam_lo), (pi_hi, ni_hi, gam_hi) = ranks[j]
        gk = gkeys[j]
        cle_lo = _fcount(gk <= klo)
        cle_hi = _fcount(gk <= khi)
        mingt_lo = jnp.min(jnp.where(gk > klo, gk, _SENT))
        mingt_hi = jnp.min(jnp.where(gk > khi, gk, _SENT))
        klo2 = jnp.where(cle_lo >= ni_lo + 1.0, klo, mingt_lo)
        khi2 = jnp.where(cle_hi >= ni_hi + 1.0, khi, mingt_hi)

        def qval(a_key, b_key, gamma):
            a = _unmap_key(a_key)
            b = _unmap_key(b_key)
            d = b - a
            return jnp.where(gamma >= 0.5, b - d * (1.0 - gamma), a + d * gamma)

        q_lo = qval(klo, klo2, gam_lo)
        q_hi = qval(khi, khi2, gam_hi)
        rng = q_hi - q_lo
        inf = np.float32(np.inf)
        ok = ((nf[j] > 0) & (jnp.abs(q_lo) < inf)
              & (jnp.abs(q_hi) < inf) & (rng > 0))
        rr = jnp.where(ok, jnp.maximum(rng, 1e-06), 1.0)
        rrs.append(rr)
        ths.append(_DIFF_RATIO * rr)
        scales.append(jnp.maximum(rr, 1e-06))

    r_ds, r_as, kds, kas, ks = [], [], [], [], []
    for j in range(_BB):
        s, t = st[j]
        pa1 = s * P[j][1] + t
        pa0 = s * P[j][0] + t
        dpred = pa1 - pa0
        curr_ref[j, 0] = dpred
        dgt = G[j][1] - G[j][0]
        static = Mb[j][1] & Mb[j][0] & (jnp.abs(dgt) < ths[j])
        n = _fcount(static)
        k = ((_KC * n) / 10.0).astype(jnp.int32).astype(jnp.float32)
        r_d = jnp.abs(dpred - dgt) / scales[j]
        r_a = jnp.abs(dpred - pv_ref[j, 0]) / scales[j]
        kd = jnp.where(static, lax.bitcast_convert_type(r_d, jnp.int32), _SENT)
        ka = jnp.where(static, lax.bitcast_convert_type(r_a, jnp.int32), _SENT)
        r_ds.append(r_d)
        r_as.append(r_a)
        kds.append(kd)
        kas.append(ka)
        ks.append(k)

    tau_keys, tau_tgts = [], []
    for j in range(_BB):
        tgt = jnp.maximum(ks[j], 1.0)
        tau_keys += [kds[j], kas[j]]
        tau_tgts += [tgt, tgt]
    tsel = _select_many(tau_keys, tau_tgts)

    lane = lax.broadcasted_iota(jnp.int32, (1, 1, 128), 2)
    for j in range(_BB):
        loss_d = _trimmed_loss(r_ds[j], kds[j], tsel[2 * j], ks[j])
        loss_a = _trimmed_loss(r_as[j], kas[j], tsel[2 * j + 1], ks[j])
        row = jnp.where(lane == 0, loss_d,
                        jnp.where(lane == 1, loss_a,
                                  jnp.where(lane == 2, rrs[j], 0.0)))
        scal_ref[j] = row[0]


@jax.jit
def kernel(pred_pair, gt_pair, mask_pair, prev_pred_grad):
    B, T, H, W = pred_pair.shape
    grid = (B // _BB,)
    scal, curr = pl.pallas_call(
        _body,
        grid=grid,
        in_specs=[
            pl.BlockSpec((_BB, 2, H, W), lambda b: (b, 0, 0, 0)),
            pl.BlockSpec((_BB, 2, H, W), lambda b: (b, 0, 0, 0)),
            pl.BlockSpec((_BB, 2, H, W), lambda b: (b, 0, 0, 0)),
            pl.BlockSpec((_BB, 1, H, W), lambda b: (b, 0, 0, 0)),
        ],
        out_specs=[
            pl.BlockSpec((_BB, 1, 128), lambda b: (b, 0, 0)),
            pl.BlockSpec((_BB, 1, H, W), lambda b: (b, 0, 0, 0)),
        ],
        out_shape=[
            jax.ShapeDtypeStruct((B, 1, 128), jnp.float32),
            jax.ShapeDtypeStruct((B, 1, H, W), jnp.float32),
        ],
    )(pred_pair, gt_pair, mask_pair, prev_pred_grad)
    loss_data = jnp.mean(scal[:, 0, 0])
    loss_acc = jnp.mean(scal[:, 0, 1])
    mean_rr = jnp.mean(scal[:, 0, 2])
    total = loss_data + _LAMBDA_ACCEL * loss_acc
    return (total, loss_data, loss_acc, mean_rr, curr)
